# baseline (speedup 1.0000x reference)
import numpy as np
import jax
import jax.numpy as jnp
from jax.experimental import pallas as pl
from jax.experimental.pallas import tpu as pltpu

N, I, S, C, K, B = 1024, 1024, 2048, 4, 16, 256
NB = 128

def _blk(w_ref, lp_ref, out_ref):
    m = jnp.dot(w_ref[:].astype(jnp.bfloat16), lp_ref[:].astype(jnp.bfloat16),
                preferred_element_type=jnp.float32)
    out_ref[:] = m[0:NB, :]

@jax.jit
def _probe(lp, w2d):
    return pl.pallas_call(
        _blk,
        grid=(N // NB,),
        in_specs=[
            pl.BlockSpec((512, I), lambda i: (i, 0)),
            pl.BlockSpec((I, B), lambda i: (0, 0)),
        ],
        out_specs=pl.BlockSpec((NB, B), lambda i: (i, 0)),
        out_shape=jax.ShapeDtypeStruct((N, B), jnp.float32),
    )(w2d, lp)

def kernel(logit_previous, side_information, v, b, weights, boolean_converter, bias):
    v2d = v.reshape(N * C, S)
    return _probe(logit_previous, v2d[:, 0:1024])


# speedup vs baseline: 1.4774x; 1.4774x over previous
import functools

import numpy as np
import jax
import jax.numpy as jnp
from jax.experimental import pallas as pl
from jax.experimental.pallas import tpu as pltpu

N = 1024
I = 1024
S = 2048
C = 4
K = 2 ** C
B = 256

NB = 128


def _lv_block(v_ref, b_ref, w_ref, si_ref, lp_ref, bias_ref, out_ref):
    v_cat = jnp.concatenate([v_ref[:, c, :] for c in range(C)], axis=0)
    proj = jnp.dot(v_cat.astype(jnp.bfloat16), si_ref[:].astype(jnp.bfloat16),
                   preferred_element_type=jnp.float32)
    ctx = jnp.zeros((NB, B), jnp.int32)
    for c in range(C):
        bit = proj[c * NB:(c + 1) * NB, :] > b_ref[:, c, :]
        ctx = ctx + jnp.where(bit, 1 << c, 0)

    m = jnp.dot(w_ref[:].astype(jnp.bfloat16), lp_ref[:].astype(jnp.bfloat16),
                preferred_element_type=jnp.float32)
    m3 = m.reshape(NB, K, B)
    kio = jax.lax.broadcasted_iota(jnp.int32, (1, K, 1), 1)
    sel = jnp.where(ctx[:, None, :] == kio, m3, 0.0)
    out_ref[:] = jnp.sum(sel, axis=1)

    @pl.when(pl.program_id(0) == 0)
    def _():
        out_ref[0:1, :] = jnp.full((1, B), bias_ref[0], jnp.float32)


@jax.jit
def _layer_vec(lp, si, v, b3, w2d, bias):
    bias_arr = jnp.reshape(bias.astype(jnp.float32), (1,))
    out = pl.pallas_call(
        _lv_block,
        grid=(N // NB,),
        in_specs=[
            pl.BlockSpec((NB, C, S), lambda i: (i, 0, 0)),
            pl.BlockSpec((NB, C, 1), lambda i: (i, 0, 0)),
            pl.BlockSpec((NB * K, I), lambda i: (i, 0)),
            pl.BlockSpec((S, B), lambda i: (0, 0)),
            pl.BlockSpec((I, B), lambda i: (0, 0)),
            pl.BlockSpec(memory_space=pltpu.SMEM),
        ],
        out_specs=pl.BlockSpec((NB, B), lambda i: (i, 0)),
        out_shape=jax.ShapeDtypeStruct((N, B), jnp.float32),
    )(v, b3, w2d, si, lp, bias_arr)
    return out


def kernel(logit_previous, side_information, v, b, weights, boolean_converter, bias):
    w2d = weights.reshape(N * K, I)
    bias_f = jnp.asarray(bias, dtype=jnp.float32)
    return _layer_vec(logit_previous, side_information, v, b, w2d, bias_f)
